# baseline (speedup 1.0000x reference)
import functools

import jax
import jax.numpy as jnp
from jax import lax
from jax.experimental import pallas as pl
from jax.experimental.pallas import tpu as pltpu
from jax.experimental.pallas import tpu_sc as plsc

D = 32
G = 128
QB = 4
U = QB * G
NW = 32
B_TOTAL = 4096 * 200
B_PER_W = B_TOTAL // NW
NU = B_PER_W // U
NT = 4096 // 128
NQ = NT // QB

_mesh = plsc.VectorSubcoreMesh(core_axis_name="c", subcore_axis_name="s")

V = 1000000
VFULL = (V // G) * G
NCH = VFULL // G
NCH_W = 246


@functools.partial(
    pl.kernel,
    out_type=jax.ShapeDtypeStruct((V * D,), jnp.float32),
    mesh=_mesh,
    scratch_types=[
        pltpu.VMEM((D, G + 1), jnp.float32),
        pltpu.VMEM((D, G + 1), jnp.float32),
        pltpu.VMEM((G * D,), jnp.float32),
        pltpu.VMEM((G * D,), jnp.float32),
        pltpu.VMEM((2048,), jnp.float32),
        pltpu.SemaphoreType.DMA((2,)),
        pltpu.SemaphoreType.DMA((2,)),
    ],
    compiler_params=pltpu.CompilerParams(
        use_tc_tiling_on_sc=True,
        needs_layout_passes=False,
        disable_bounds_checks=True,
    ),
)
def _detile(tt_hbm, tail_hbm, out_hbm, stg0, stg1, lin0, lin1, tail_v, ssem, wsem):
    wid = lax.axis_index("s") * 2 + lax.axis_index("c")
    iota = lax.iota(jnp.int32, 16)
    rowsel = [(dh * 16) + iota for dh in range(2)]
    stg_b = (stg0, stg1)
    lin_b = (lin0, lin1)

    def fire_stage(c, p):
        pltpu.async_copy(
            tt_hbm.at[pl.ds(0, D), pl.ds(c * G, G)],
            stg_b[p].at[pl.ds(0, D), pl.ds(0, G)],
            ssem.at[p]
        )

    def wait_stage(p):
        pltpu.make_async_copy(
            tt_hbm.at[pl.ds(0, D), pl.ds(0, G)],
            stg_b[p].at[pl.ds(0, D), pl.ds(0, G)],
            ssem.at[p]
        ).wait()

    def transpose(p):
        stg = stg_b[p]
        dst = lin_b[p]

        @plsc.parallel_loop(0, G, unroll=8)
        def _v(v):
            col = jnp.full((16,), v, jnp.int32)
            for dh in range(2):
                vec = plsc.load_gather(stg, [rowsel[dh], col])
                dst[pl.ds(v * D + dh * 16, 16)] = vec

    def fire_write(c, p):
        pltpu.async_copy(
            lin_b[p], out_hbm.at[pl.ds(c * G * D, G * D)], wsem.at[p]
        )

    def wait_write(p):
        pltpu.make_async_copy(
            lin_b[p], out_hbm.at[pl.ds(0, G * D)], wsem.at[p]
        ).wait()

    @pl.when(wid == 0)
    def _():
        pltpu.sync_copy(tail_hbm, tail_v)
        pltpu.sync_copy(tail_v, out_hbm.at[pl.ds(VFULL * D, (V - VFULL) * D)])

    @pl.when(wid < NCH)
    def _():
        fire_stage(wid, 0)

    @pl.when(wid + NW < NCH)
    def _():
        fire_stage(wid + NW, 1)

    @pl.loop(0, NCH_W, step=2)
    def _step(k):
        c0 = wid + k * NW

        @pl.when(c0 < NCH)
        def _():
            wait_stage(0)
            @pl.when(k > 0)
            def _():
                wait_write(0)
            transpose(0)
            fire_write(c0, 0)
            @pl.when(c0 + 2 * NW < NCH)
            def _():
                fire_stage(c0 + 2 * NW, 0)

        @pl.when(c0 + NW < NCH)
        def _():
            wait_stage(1)
            @pl.when(k > 0)
            def _():
                wait_write(1)
            transpose(1)
            fire_write(c0 + NW, 1)
            @pl.when(c0 + 3 * NW < NCH)
            def _():
                fire_stage(c0 + 3 * NW, 1)

    @pl.when(wid < NCH)
    def _():
        wait_write(0)

    @pl.when(wid + NW < NCH)
    def _():
        wait_write(1)


@functools.partial(
    pl.kernel,
    out_type=jax.ShapeDtypeStruct((200, 4, NT, 8, 128), jnp.float32),
    mesh=_mesh,
    scratch_types=[
        pltpu.VMEM((B_PER_W,), jnp.int32),
        pltpu.VMEM((2, U, D), jnp.float32),
        pltpu.VMEM((2, 4, QB, 8, G), jnp.float32),
        pltpu.SemaphoreType.DMA((2,)),
        pltpu.SemaphoreType.DMA((2,)),
    ],
    compiler_params=pltpu.CompilerParams(
        use_tc_tiling_on_sc=False,
        needs_layout_passes=False,
        disable_bounds_checks=True,
    ),
)
def _emb_lookup(xs_hbm, table_hbm, out_hbm, idx_v, rows_v, t_v, gsem, wsem):
    wid = lax.axis_index("s") * 2 + lax.axis_index("c")
    q_base = wid * NU
    iota = lax.iota(jnp.int32, 16)

    pltpu.sync_copy(xs_hbm.at[pl.ds(q_base * U, B_PER_W)], idx_v)

    def fire_gathers(ci, p):
        for s in range(QB):
            pltpu.async_copy(
                table_hbm.at[idx_v.at[pl.ds(ci * U + s * G, G)]],
                rows_v.at[p].at[pl.ds(s * G, G)],
                gsem.at[p],
            )

    def wait_gathers(p):
        pltpu.make_async_copy(
            table_hbm.at[idx_v.at[pl.ds(0, U)]], rows_v.at[p], gsem.at[p]
        ).wait()

    rowsel = [
        [iota + (bcs * G + b0 * 16) for b0 in range(G // 16)] for bcs in range(QB)
    ]

    def transpose(p):
        rows = rows_v.at[p]
        dst4 = t_v.at[p]

        @plsc.parallel_loop(0, D, unroll=8)
        def _d(d):
            tr = d // 8
            dd = lax.rem(d, 8)
            col = jnp.full((16,), d, jnp.int32)
            for bcs in range(QB):
                for b0 in range(G // 16):
                    vec = plsc.load_gather(rows, [rowsel[bcs][b0], col])
                    dst4[tr, bcs, dd, pl.ds(b0 * 16, 16)] = vec

    def fire_writes(ci, p):
        q = q_base + ci
        t = q // NQ
        bc0 = lax.rem(q, NQ) * QB
        for tr in range(4):
            pltpu.async_copy(
                t_v.at[p].at[tr], out_hbm.at[t, tr, pl.ds(bc0, QB)], wsem.at[p]
            )

    def wait_writes(p):
        pltpu.make_async_copy(
            t_v.at[p], out_hbm.at[0].at[:, pl.ds(0, QB)], wsem.at[p]
        ).wait()

    fire_gathers(0, 0)
    fire_gathers(1, 1)

    @pl.loop(0, NU, step=2)
    def _step(ci):
        wait_gathers(0)
        @pl.when(ci > 0)
        def _():
            wait_writes(0)
        transpose(0)
        fire_writes(ci, 0)
        @pl.when(ci + 2 < NU)
        def _():
            fire_gathers(ci + 2, 0)
        wait_gathers(1)
        @pl.when(ci > 0)
        def _():
            wait_writes(1)
        transpose(1)
        fire_writes(ci + 1, 1)
        @pl.when(ci + 3 < NU)
        def _():
            fire_gathers(ci + 3, 1)

    wait_writes(0)
    wait_writes(1)


def kernel(xs, table):
    tail = table[VFULL:].reshape((V - VFULL) * D)
    table_lin = _detile(table.T, tail).reshape(V, D)
    out5 = _emb_lookup(xs.T.reshape(B_TOTAL), table_lin)
    return out5.transpose(2, 4, 0, 1, 3).reshape(4096, 200, D)


# speedup vs baseline: 1.0690x; 1.0690x over previous
import functools

import jax
import jax.numpy as jnp
from jax import lax
from jax.experimental import pallas as pl
from jax.experimental.pallas import tpu as pltpu
from jax.experimental.pallas import tpu_sc as plsc

D = 32
G = 128
QB = 4
U = QB * G
NW = 32
B_TOTAL = 4096 * 200
B_PER_W = B_TOTAL // NW
NU = B_PER_W // U
NT = 4096 // 128
NQ = NT // QB

_mesh = plsc.VectorSubcoreMesh(core_axis_name="c", subcore_axis_name="s")

V = 1000000
VFULL = (V // G) * G
NCH = VFULL // G
NCH_W = 246


@functools.partial(
    pl.kernel,
    out_type=jax.ShapeDtypeStruct((V * D,), jnp.float32),
    mesh=_mesh,
    scratch_types=[
        pltpu.VMEM((D, G + 1), jnp.float32),
        pltpu.VMEM((D, G + 1), jnp.float32),
        pltpu.VMEM((G * D,), jnp.float32),
        pltpu.VMEM((G * D,), jnp.float32),
        pltpu.VMEM((2048,), jnp.float32),
        pltpu.SemaphoreType.DMA((2,)),
        pltpu.SemaphoreType.DMA((2,)),
    ],
    compiler_params=pltpu.CompilerParams(
        use_tc_tiling_on_sc=True,
        needs_layout_passes=False,
        disable_bounds_checks=True,
    ),
)
def _detile(tt_hbm, tail_hbm, out_hbm, stg0, stg1, lin0, lin1, tail_v, ssem, wsem):
    wid = lax.axis_index("s") * 2 + lax.axis_index("c")
    iota = lax.iota(jnp.int32, 16)
    rowsel = [(dh * 16) + iota for dh in range(2)]
    stg_b = (stg0, stg1)
    lin_b = (lin0, lin1)

    def fire_stage(c, p):
        pltpu.async_copy(
            tt_hbm.at[pl.ds(0, D), pl.ds(c * G, G)],
            stg_b[p].at[pl.ds(0, D), pl.ds(0, G)],
            ssem.at[p]
        )

    def wait_stage(p):
        pltpu.make_async_copy(
            tt_hbm.at[pl.ds(0, D), pl.ds(0, G)],
            stg_b[p].at[pl.ds(0, D), pl.ds(0, G)],
            ssem.at[p]
        ).wait()

    def transpose(p):
        stg = stg_b[p]
        dst = lin_b[p]

        @plsc.parallel_loop(0, G, unroll=4)
        def _v(v):
            col = jnp.full((16,), v, jnp.int32)
            for dh in range(2):
                vec = plsc.load_gather(stg, [rowsel[dh], col])
                dst[pl.ds(v * D + dh * 16, 16)] = vec

    def fire_write(c, p):
        pltpu.async_copy(
            lin_b[p], out_hbm.at[pl.ds(c * G * D, G * D)], wsem.at[p]
        )

    def wait_write(p):
        pltpu.make_async_copy(
            lin_b[p], out_hbm.at[pl.ds(0, G * D)], wsem.at[p]
        ).wait()

    @pl.when(wid == 0)
    def _():
        pltpu.sync_copy(tail_hbm, tail_v)
        pltpu.sync_copy(tail_v, out_hbm.at[pl.ds(VFULL * D, (V - VFULL) * D)])

    @pl.when(wid < NCH)
    def _():
        fire_stage(wid, 0)

    @pl.when(wid + NW < NCH)
    def _():
        fire_stage(wid + NW, 1)

    @pl.loop(0, NCH_W, step=2)
    def _step(k):
        c0 = wid + k * NW

        @pl.when(c0 < NCH)
        def _():
            wait_stage(0)
            @pl.when(k > 0)
            def _():
                wait_write(0)
            transpose(0)
            fire_write(c0, 0)
            @pl.when(c0 + 2 * NW < NCH)
            def _():
                fire_stage(c0 + 2 * NW, 0)

        @pl.when(c0 + NW < NCH)
        def _():
            wait_stage(1)
            @pl.when(k > 0)
            def _():
                wait_write(1)
            transpose(1)
            fire_write(c0 + NW, 1)
            @pl.when(c0 + 3 * NW < NCH)
            def _():
                fire_stage(c0 + 3 * NW, 1)

    @pl.when(wid < NCH)
    def _():
        wait_write(0)

    @pl.when(wid + NW < NCH)
    def _():
        wait_write(1)


@functools.partial(
    pl.kernel,
    out_type=jax.ShapeDtypeStruct((200, 4, NT, 8, 128), jnp.float32),
    mesh=_mesh,
    scratch_types=[
        pltpu.VMEM((B_PER_W,), jnp.int32),
        pltpu.VMEM((2, U, D), jnp.float32),
        pltpu.VMEM((2, 4, QB, 8, G), jnp.float32),
        pltpu.SemaphoreType.DMA((2,)),
        pltpu.SemaphoreType.DMA((2,)),
    ],
    compiler_params=pltpu.CompilerParams(
        use_tc_tiling_on_sc=False,
        needs_layout_passes=False,
        disable_bounds_checks=True,
    ),
)
def _emb_lookup(xs_hbm, table_hbm, out_hbm, idx_v, rows_v, t_v, gsem, wsem):
    wid = lax.axis_index("s") * 2 + lax.axis_index("c")
    q_base = wid * NU
    iota = lax.iota(jnp.int32, 16)

    pltpu.sync_copy(xs_hbm.at[pl.ds(q_base * U, B_PER_W)], idx_v)

    def fire_gathers(ci, p):
        for s in range(QB):
            pltpu.async_copy(
                table_hbm.at[idx_v.at[pl.ds(ci * U + s * G, G)]],
                rows_v.at[p].at[pl.ds(s * G, G)],
                gsem.at[p],
            )

    def wait_gathers(p):
        pltpu.make_async_copy(
            table_hbm.at[idx_v.at[pl.ds(0, U)]], rows_v.at[p], gsem.at[p]
        ).wait()

    rowsel = [
        [iota + (bcs * G + b0 * 16) for b0 in range(G // 16)] for bcs in range(QB)
    ]

    def transpose(p):
        rows = rows_v.at[p]
        for tr in range(4):
            dst = t_v.at[p].at[tr]

            @plsc.parallel_loop(0, 8, unroll=4)
            def _dd(dd):
                col = jnp.full((16,), tr * 8 + dd, jnp.int32)
                for bcs in range(QB):
                    for b0 in range(G // 16):
                        vec = plsc.load_gather(rows, [rowsel[bcs][b0], col])
                        dst[bcs, dd, pl.ds(b0 * 16, 16)] = vec

    def fire_writes(ci, p):
        q = q_base + ci
        t = q // NQ
        bc0 = lax.rem(q, NQ) * QB
        for tr in range(4):
            pltpu.async_copy(
                t_v.at[p].at[tr], out_hbm.at[t, tr, pl.ds(bc0, QB)], wsem.at[p]
            )

    def wait_writes(p):
        pltpu.make_async_copy(
            t_v.at[p], out_hbm.at[0].at[:, pl.ds(0, QB)], wsem.at[p]
        ).wait()

    fire_gathers(0, 0)
    fire_gathers(1, 1)

    @pl.loop(0, NU, step=2)
    def _step(ci):
        wait_gathers(0)
        @pl.when(ci > 0)
        def _():
            wait_writes(0)
        transpose(0)
        fire_writes(ci, 0)
        @pl.when(ci + 2 < NU)
        def _():
            fire_gathers(ci + 2, 0)
        wait_gathers(1)
        @pl.when(ci > 0)
        def _():
            wait_writes(1)
        transpose(1)
        fire_writes(ci + 1, 1)
        @pl.when(ci + 3 < NU)
        def _():
            fire_gathers(ci + 3, 1)

    wait_writes(0)
    wait_writes(1)


def kernel(xs, table):
    tail = table[VFULL:].reshape((V - VFULL) * D)
    table_lin = _detile(table.T, tail).reshape(V, D)
    out5 = _emb_lookup(xs.T.reshape(B_TOTAL), table_lin)
    return out5.transpose(2, 4, 0, 1, 3).reshape(4096, 200, D)
